# baseline (speedup 1.0000x reference)
import functools

import jax
import jax.numpy as jnp
from jax import lax
from jax.experimental import pallas as pl
from jax.experimental.pallas import tpu as pltpu
from jax.experimental.pallas import tpu_sc as plsc

E = 8
K = 2
ROW_BLOCK = 128
NC = 2
NS = 16
NW = NC * NS
LANES = 16


def _moe_gemm_body(bid_ref, eidx_ref, ecmp_ref,
                   x_ref, w1_ref, b1_ref, w2_ref, b2_ref, es_ref, gs_ref,
                   out_ref):
    s = pl.program_id(0)
    bid = bid_ref[s]
    prev = bid_ref[jnp.maximum(s - 1, 0)]
    first_visit = jnp.logical_or(s == 0, bid != prev)

    @pl.when(first_visit)
    def _():
        out_ref[...] = jnp.zeros_like(out_ref)

    xb = x_ref[...]
    w1 = w1_ref[0]
    b1 = b1_ref[0]
    h = jnp.dot(xb, w1, preferred_element_type=jnp.float32) + b1
    h = jnp.maximum(h, 0.0)
    w2 = w2_ref[0]
    y = jnp.dot(h, w2, preferred_element_type=jnp.float32) + b2_ref[0]

    e_cmp = ecmp_ref[s]
    scale = jnp.where(es_ref[0, 0, :] == e_cmp, gs_ref[0, 0, :], 0.0)
    out_ref[...] += y * scale[:, None]


def _grouped_gemm(meta, x_sorted, W1, b1, W2, b2, es, gs, n_rows, d_model,
                  d_ff):
    bid, eidx, ecmp = meta
    nb = n_rows // ROW_BLOCK
    g = nb + E - 1
    grid_spec = pltpu.PrefetchScalarGridSpec(
        num_scalar_prefetch=3,
        grid=(g,),
        in_specs=[
            pl.BlockSpec((ROW_BLOCK, d_model),
                         lambda s, bid, eidx, ecmp: (bid[s], 0)),
            pl.BlockSpec((1, d_model, d_ff),
                         lambda s, bid, eidx, ecmp: (eidx[s], 0, 0)),
            pl.BlockSpec((1, 1, d_ff),
                         lambda s, bid, eidx, ecmp: (eidx[s], 0, 0)),
            pl.BlockSpec((1, d_ff, d_model),
                         lambda s, bid, eidx, ecmp: (eidx[s], 0, 0)),
            pl.BlockSpec((1, 1, d_model),
                         lambda s, bid, eidx, ecmp: (eidx[s], 0, 0)),
            pl.BlockSpec((1, 1, ROW_BLOCK),
                         lambda s, bid, eidx, ecmp: (bid[s], 0, 0)),
            pl.BlockSpec((1, 1, ROW_BLOCK),
                         lambda s, bid, eidx, ecmp: (bid[s], 0, 0)),
        ],
        out_specs=pl.BlockSpec((ROW_BLOCK, d_model),
                               lambda s, bid, eidx, ecmp: (bid[s], 0)),
    )
    return pl.pallas_call(
        _moe_gemm_body,
        grid_spec=grid_spec,
        out_shape=jax.ShapeDtypeStruct((n_rows, d_model), jnp.float32),
        compiler_params=pltpu.CompilerParams(
            dimension_semantics=("arbitrary",)),
    )(bid, eidx, ecmp,
      x_sorted, W1, b1[:, None, :], W2, b2[:, None, :],
      es.reshape(nb, 1, ROW_BLOCK), gs.reshape(nb, 1, ROW_BLOCK))


def _make_dispatch_gather(n_tok, d_model, n_rows):
    rows_per_w = n_rows // NW
    gath = 32
    nch = rows_per_w // gath
    mesh = plsc.VectorSubcoreMesh(core_axis_name="c", subcore_axis_name="s")

    @functools.partial(
        pl.kernel, mesh=mesh,
        out_type=jax.ShapeDtypeStruct((n_rows, d_model), jnp.float32),
        scratch_types=[
            pltpu.VMEM((nch, gath), jnp.int32),
            pltpu.VMEM((2, gath, d_model), jnp.float32),
            pltpu.SemaphoreType.DMA,
            pltpu.SemaphoreType.DMA,
        ],
    )
    def dispatch_gather(tok_hbm, x_hbm, xs_hbm, tk_loc, rows_v, sem_g, sem_p):
        wid = lax.axis_index("s") * NC + lax.axis_index("c")
        own_lo = wid * rows_per_w
        for c in range(nch):
            pltpu.sync_copy(tok_hbm.at[pl.ds(own_lo + c * gath, gath)],
                            tk_loc.at[c])
        gets = []
        for c in range(nch):
            gets.append(pltpu.async_copy(
                x_hbm.at[tk_loc.at[c]], rows_v.at[c % 2], sem_g))
            if c > 0:
                gets[c - 1].wait()
                pltpu.async_copy(
                    rows_v.at[(c - 1) % 2],
                    xs_hbm.at[pl.ds(own_lo + (c - 1) * gath, gath)],
                    sem_p).wait()
        gets[nch - 1].wait()
        pltpu.sync_copy(rows_v.at[(nch - 1) % 2],
                        xs_hbm.at[pl.ds(own_lo + (nch - 1) * gath, gath)])

    return dispatch_gather


def _make_combine(n_tok, d_model, n_rows):
    tok_per_w = n_tok // NW
    gath = 64
    mesh = plsc.VectorSubcoreMesh(core_axis_name="c", subcore_axis_name="s")

    @functools.partial(
        pl.kernel, mesh=mesh,
        out_type=jax.ShapeDtypeStruct((n_tok, d_model), jnp.float32),
        scratch_types=[
            pltpu.VMEM((K * tok_per_w // gath, gath), jnp.int32),
            pltpu.VMEM((gath, d_model), jnp.float32),
            pltpu.VMEM((gath // K, d_model), jnp.float32),
            pltpu.SemaphoreType.DMA,
        ],
    )
    def combine(y_hbm, inv_hbm, out_hbm, iv_v, rows_v, out_v, sem):
        wid = lax.axis_index("s") * NC + lax.axis_index("c")
        base = wid * K * tok_per_w
        for c in range(K * tok_per_w // gath):
            pltpu.sync_copy(inv_hbm.at[pl.ds(base + c * gath, gath)],
                            iv_v.at[c])
            pltpu.async_copy(y_hbm.at[iv_v.at[c]], rows_v, sem).wait()

            def r_body(i, carry):
                for dch in range(d_model // LANES):
                    sl = pl.ds(dch * LANES, LANES)
                    out_v[i, sl] = rows_v[2 * i, sl] + rows_v[2 * i + 1, sl]
                return carry

            lax.fori_loop(0, gath // K, r_body, 0)
            pltpu.sync_copy(
                out_v,
                out_hbm.at[pl.ds(wid * tok_per_w + c * (gath // K),
                                 gath // K)])

    return combine


def kernel(x, Wg, W1, b1, W2, b2):
    n_tok, d_model = x.shape
    d_ff = W1.shape[2]
    n_rows = n_tok * K

    logits = x @ Wg
    top_vals, top_idx = jax.lax.top_k(logits, K)
    gz = top_vals - top_vals[:, :1]
    ez = jnp.exp(gz)
    gates = ez / jnp.sum(ez, axis=-1, keepdims=True)

    eflat = top_idx.reshape(-1).astype(jnp.int32)
    gflat = gates.reshape(-1)

    nb = n_rows // ROW_BLOCK
    g_steps = nb + E - 1
    order = jnp.argsort(eflat, stable=True)
    es = eflat[order]
    gs = gflat[order]
    tok_sorted = (order // K).astype(jnp.int32)
    inv = jnp.zeros((n_rows,), jnp.int32).at[order].set(
        jnp.arange(n_rows, dtype=jnp.int32))

    dispatch_gather = _make_dispatch_gather(n_tok, d_model, n_rows)
    x_sorted = dispatch_gather(tok_sorted, x)

    blk = es.reshape(nb, ROW_BLOCK)
    fe = blk[:, 0]
    le = blk[:, -1]
    spb = le - fe + 1
    step_start = jnp.concatenate(
        [jnp.zeros((1,), jnp.int32), jnp.cumsum(spb)[:-1]])
    total = step_start[-1] + spb[-1]
    s_ar = jnp.arange(g_steps, dtype=jnp.int32)
    i_of_s = jnp.searchsorted(step_start, s_ar, side='right') - 1
    i_of_s = jnp.clip(i_of_s, 0, nb - 1).astype(jnp.int32)
    e_of_s = fe[i_of_s] + (s_ar - step_start[i_of_s])
    valid = s_ar < total
    bid = jnp.where(valid, i_of_s, nb - 1).astype(jnp.int32)
    eidx = jnp.clip(e_of_s, 0, E - 1).astype(jnp.int32)
    ecmp = jnp.where(valid, e_of_s, -1).astype(jnp.int32)

    y_sorted = _grouped_gemm((bid, eidx, ecmp), x_sorted,
                             W1, b1, W2, b2, es, gs, n_rows, d_model, d_ff)

    combine = _make_combine(n_tok, d_model, n_rows)
    return combine(y_sorted, inv)


def _xla_combine(y_sorted, inv, n_tok):
    pos = inv.reshape(n_tok, K)
    out = jnp.take(y_sorted, pos[:, 0], axis=0)
    for k in range(1, K):
        out = out + jnp.take(y_sorted, pos[:, k], axis=0)
    return out


# speedup vs baseline: 1.0980x; 1.0980x over previous
import functools

import jax
import jax.numpy as jnp
from jax import lax
from jax.experimental import pallas as pl
from jax.experimental.pallas import tpu as pltpu
from jax.experimental.pallas import tpu_sc as plsc

E = 8
K = 2
ROW_BLOCK = 512
NC = 2
NS = 16
NW = NC * NS
LANES = 16


def _moe_gemm_body(bid_ref, eidx_ref, ecmp_ref,
                   x_ref, w1_ref, b1_ref, w2_ref, b2_ref, es_ref, gs_ref,
                   out_ref):
    s = pl.program_id(0)
    bid = bid_ref[s]
    prev = bid_ref[jnp.maximum(s - 1, 0)]
    first_visit = jnp.logical_or(s == 0, bid != prev)

    @pl.when(first_visit)
    def _():
        out_ref[...] = jnp.zeros_like(out_ref)

    xb = x_ref[...]
    w1 = w1_ref[0]
    b1 = b1_ref[0]
    h = jnp.dot(xb, w1, preferred_element_type=jnp.float32) + b1
    h = jnp.maximum(h, 0.0)
    w2 = w2_ref[0]
    y = jnp.dot(h, w2, preferred_element_type=jnp.float32) + b2_ref[0]

    e_cmp = ecmp_ref[s]
    scale = jnp.where(es_ref[0, 0, :] == e_cmp, gs_ref[0, 0, :], 0.0)
    out_ref[...] += y * scale[:, None]


def _grouped_gemm(meta, x_sorted, W1, b1, W2, b2, es, gs, n_rows, d_model,
                  d_ff):
    bid, eidx, ecmp = meta
    nb = n_rows // ROW_BLOCK
    g = nb + E - 1
    grid_spec = pltpu.PrefetchScalarGridSpec(
        num_scalar_prefetch=3,
        grid=(g,),
        in_specs=[
            pl.BlockSpec((ROW_BLOCK, d_model),
                         lambda s, bid, eidx, ecmp: (bid[s], 0)),
            pl.BlockSpec((1, d_model, d_ff),
                         lambda s, bid, eidx, ecmp: (eidx[s], 0, 0)),
            pl.BlockSpec((1, 1, d_ff),
                         lambda s, bid, eidx, ecmp: (eidx[s], 0, 0)),
            pl.BlockSpec((1, d_ff, d_model),
                         lambda s, bid, eidx, ecmp: (eidx[s], 0, 0)),
            pl.BlockSpec((1, 1, d_model),
                         lambda s, bid, eidx, ecmp: (eidx[s], 0, 0)),
            pl.BlockSpec((1, 1, ROW_BLOCK),
                         lambda s, bid, eidx, ecmp: (bid[s], 0, 0)),
            pl.BlockSpec((1, 1, ROW_BLOCK),
                         lambda s, bid, eidx, ecmp: (bid[s], 0, 0)),
        ],
        out_specs=pl.BlockSpec((ROW_BLOCK, d_model),
                               lambda s, bid, eidx, ecmp: (bid[s], 0)),
    )
    return pl.pallas_call(
        _moe_gemm_body,
        grid_spec=grid_spec,
        out_shape=jax.ShapeDtypeStruct((n_rows, d_model), jnp.float32),
        compiler_params=pltpu.CompilerParams(
            dimension_semantics=("arbitrary",)),
    )(bid, eidx, ecmp,
      x_sorted, W1, b1[:, None, :], W2, b2[:, None, :],
      es.reshape(nb, 1, ROW_BLOCK), gs.reshape(nb, 1, ROW_BLOCK))


def _make_dispatch_gather(n_tok, d_model, n_rows):
    rows_per_w = n_rows // NW
    gath = 32
    nch = rows_per_w // gath
    mesh = plsc.VectorSubcoreMesh(core_axis_name="c", subcore_axis_name="s")

    @functools.partial(
        pl.kernel, mesh=mesh,
        out_type=jax.ShapeDtypeStruct((n_rows, d_model), jnp.float32),
        scratch_types=[
            pltpu.VMEM((nch, gath), jnp.int32),
            pltpu.VMEM((2, gath, d_model), jnp.float32),
            pltpu.SemaphoreType.DMA,
            pltpu.SemaphoreType.DMA,
        ],
    )
    def dispatch_gather(tok_hbm, x_hbm, xs_hbm, tk_loc, rows_v, sem_g, sem_p):
        wid = lax.axis_index("s") * NC + lax.axis_index("c")
        own_lo = wid * rows_per_w
        for c in range(nch):
            pltpu.sync_copy(tok_hbm.at[pl.ds(own_lo + c * gath, gath)],
                            tk_loc.at[c])
        gets = []
        for c in range(nch):
            gets.append(pltpu.async_copy(
                x_hbm.at[tk_loc.at[c]], rows_v.at[c % 2], sem_g))
            if c > 0:
                gets[c - 1].wait()
                pltpu.async_copy(
                    rows_v.at[(c - 1) % 2],
                    xs_hbm.at[pl.ds(own_lo + (c - 1) * gath, gath)],
                    sem_p).wait()
        gets[nch - 1].wait()
        pltpu.sync_copy(rows_v.at[(nch - 1) % 2],
                        xs_hbm.at[pl.ds(own_lo + (nch - 1) * gath, gath)])

    return dispatch_gather


def _make_combine(n_tok, d_model, n_rows):
    tok_per_w = n_tok // NW
    gath = 64
    mesh = plsc.VectorSubcoreMesh(core_axis_name="c", subcore_axis_name="s")

    @functools.partial(
        pl.kernel, mesh=mesh,
        out_type=jax.ShapeDtypeStruct((n_tok, d_model), jnp.float32),
        scratch_types=[
            pltpu.VMEM((K * tok_per_w // gath, gath), jnp.int32),
            pltpu.VMEM((gath, d_model), jnp.float32),
            pltpu.VMEM((gath // K, d_model), jnp.float32),
            pltpu.SemaphoreType.DMA,
        ],
    )
    def combine(y_hbm, inv_hbm, out_hbm, iv_v, rows_v, out_v, sem):
        wid = lax.axis_index("s") * NC + lax.axis_index("c")
        base = wid * K * tok_per_w
        for c in range(K * tok_per_w // gath):
            pltpu.sync_copy(inv_hbm.at[pl.ds(base + c * gath, gath)],
                            iv_v.at[c])
            pltpu.async_copy(y_hbm.at[iv_v.at[c]], rows_v, sem).wait()

            def r_body(i, carry):
                for dch in range(d_model // LANES):
                    sl = pl.ds(dch * LANES, LANES)
                    out_v[i, sl] = rows_v[2 * i, sl] + rows_v[2 * i + 1, sl]
                return carry

            lax.fori_loop(0, gath // K, r_body, 0)
            pltpu.sync_copy(
                out_v,
                out_hbm.at[pl.ds(wid * tok_per_w + c * (gath // K),
                                 gath // K)])

    return combine


def kernel(x, Wg, W1, b1, W2, b2):
    n_tok, d_model = x.shape
    d_ff = W1.shape[2]
    n_rows = n_tok * K

    logits = x @ Wg
    top_vals, top_idx = jax.lax.top_k(logits, K)
    gz = top_vals - top_vals[:, :1]
    ez = jnp.exp(gz)
    gates = ez / jnp.sum(ez, axis=-1, keepdims=True)

    eflat = top_idx.reshape(-1).astype(jnp.int32)
    gflat = gates.reshape(-1)

    nb = n_rows // ROW_BLOCK
    g_steps = nb + E - 1
    order = jnp.argsort(eflat, stable=True)
    es = eflat[order]
    gs = gflat[order]
    tok_sorted = (order // K).astype(jnp.int32)
    inv = jnp.zeros((n_rows,), jnp.int32).at[order].set(
        jnp.arange(n_rows, dtype=jnp.int32))

    dispatch_gather = _make_dispatch_gather(n_tok, d_model, n_rows)
    x_sorted = dispatch_gather(tok_sorted, x)

    blk = es.reshape(nb, ROW_BLOCK)
    fe = blk[:, 0]
    le = blk[:, -1]
    spb = le - fe + 1
    step_start = jnp.concatenate(
        [jnp.zeros((1,), jnp.int32), jnp.cumsum(spb)[:-1]])
    total = step_start[-1] + spb[-1]
    s_ar = jnp.arange(g_steps, dtype=jnp.int32)
    i_of_s = jnp.searchsorted(step_start, s_ar, side='right') - 1
    i_of_s = jnp.clip(i_of_s, 0, nb - 1).astype(jnp.int32)
    e_of_s = fe[i_of_s] + (s_ar - step_start[i_of_s])
    valid = s_ar < total
    bid = jnp.where(valid, i_of_s, nb - 1).astype(jnp.int32)
    eidx = jnp.clip(e_of_s, 0, E - 1).astype(jnp.int32)
    ecmp = jnp.where(valid, e_of_s, -1).astype(jnp.int32)

    y_sorted = _grouped_gemm((bid, eidx, ecmp), x_sorted,
                             W1, b1, W2, b2, es, gs, n_rows, d_model, d_ff)

    combine = _make_combine(n_tok, d_model, n_rows)
    return combine(y_sorted, inv)


def _xla_combine(y_sorted, inv, n_tok):
    pos = inv.reshape(n_tok, K)
    out = jnp.take(y_sorted, pos[:, 0], axis=0)
    for k in range(1, K):
        out = out + jnp.take(y_sorted, pos[:, k], axis=0)
    return out
